# baseline (speedup 1.0000x reference)
import functools

import jax
import jax.numpy as jnp
from jax import lax
from jax.experimental import pallas as pl
from jax.experimental.pallas import tpu as pltpu
from jax.experimental.pallas import tpu_sc as plsc

N = 10000
D = 128
E = 320000
EPS = 1e-5

NC = 2
NS = 16
NW = NC * NS
EPW = E // NW
K = 112
NCH = 91
EPWP = NCH * K
NPAD = 10240
RPT = NPAD // NS

RB = 640
GRID = NPAD // RB


def _sc_degree(comb):
    mesh = plsc.VectorSubcoreMesh(core_axis_name="c", subcore_axis_name="s")

    @functools.partial(
        pl.kernel,
        out_type=jax.ShapeDtypeStruct((NC, NPAD), jnp.float32),
        mesh=mesh,
        scratch_types=[
            pltpu.VMEM((NCH, 2, K), jnp.int32),
            pltpu.VMEM((K,), jnp.float32),
            pltpu.VMEM((RPT,), jnp.float32),
            pltpu.VMEM_SHARED((NPAD,), jnp.float32),
            pltpu.SemaphoreType.DMA,
        ],
    )
    def deg_kernel(comb_hbm, deg_hbm, idx_v, ones_v, z_v, deg_sh, sem):
        c = lax.axis_index("c")
        s = lax.axis_index("s")
        wid = s * NC + c

        def zfill(i, _):
            z_v[pl.ds(i * 16, 16)] = jnp.zeros((16,), jnp.float32)
            return 0

        lax.fori_loop(0, RPT // 16, zfill, 0)
        pltpu.sync_copy(z_v, deg_sh.at[pl.ds(s * RPT, RPT)])

        def ofill(i, _):
            ones_v[pl.ds(i * 16, 16)] = jnp.ones((16,), jnp.float32)
            return 0

        lax.fori_loop(0, K // 16, ofill, 0)
        pltpu.sync_copy(comb_hbm.at[wid], idx_v)
        plsc.subcore_barrier()

        def fire(j, _):
            pltpu.async_copy(ones_v, deg_sh.at[idx_v.at[j, 1]], sem, add=True)
            return 0

        lax.fori_loop(0, NCH, fire, 0)

        def drain(j, _):
            pltpu.make_async_copy(
                ones_v, deg_sh.at[idx_v.at[j, 1]], sem).wait()
            return 0

        lax.fori_loop(0, NCH, drain, 0)
        plsc.subcore_barrier()
        pltpu.sync_copy(deg_sh.at[pl.ds(s * RPT, RPT)],
                        deg_hbm.at[c, pl.ds(s * RPT, RPT)])

    return deg_kernel(comb)


def _sc_scatter(hts, comb):
    mesh = plsc.VectorSubcoreMesh(core_axis_name="c", subcore_axis_name="s")

    @functools.partial(
        pl.kernel,
        out_type=jax.ShapeDtypeStruct((NC, NPAD, D), jnp.float32),
        mesh=mesh,
        scratch_types=[
            pltpu.VMEM((4, 2, K), jnp.int32),
            pltpu.VMEM((2, K, D), jnp.float32),
            pltpu.SemaphoreType.DMA,
            pltpu.SemaphoreType.DMA,
            pltpu.SemaphoreType.DMA,
            pltpu.SemaphoreType.DMA,
            pltpu.SemaphoreType.DMA,
            pltpu.SemaphoreType.DMA,
            pltpu.SemaphoreType.DMA,
            pltpu.SemaphoreType.DMA,
            pltpu.VMEM_SHARED((NPAD, D), jnp.float32),
        ],
    )
    def edge_kernel(hts_hbm, comb_hbm, agg_hbm,
                    idx_v, buf_v, si0, si1, si2, si3, sg0, sg1, ss0, ss1,
                    acc_sh):
        c = lax.axis_index("c")
        s = lax.axis_index("s")
        wid = s * NC + c
        sis = (si0, si1, si2, si3)
        sgs = (sg0, sg1)
        sss = (ss0, ss1)

        def zfill(i, _):
            buf_v[0, i // 8, pl.ds((i % 8) * 16, 16)] = \
                jnp.zeros((16,), jnp.float32)
            return 0

        lax.fori_loop(0, K * (D // 16), zfill, 0)

        def zcopy(m, _):
            pltpu.sync_copy(buf_v.at[0],
                            acc_sh.at[pl.ds(s * RPT + m * K, K), :])
            return 0

        lax.fori_loop(0, RPT // K, zcopy, 0)
        pltpu.sync_copy(buf_v.at[0, pl.ds(0, RPT - (RPT // K) * K), :],
                        acc_sh.at[pl.ds(s * RPT + (RPT // K) * K,
                                        RPT - (RPT // K) * K), :])
        plsc.subcore_barrier()

        def idx_issue(j, slot):
            pltpu.async_copy(comb_hbm.at[wid, j], idx_v.at[slot], sis[slot])

        def idx_wait(j, slot):
            pltpu.make_async_copy(comb_hbm.at[wid, j], idx_v.at[slot],
                                  sis[slot]).wait()

        def g_issue(j, slot, b):
            pltpu.async_copy(hts_hbm.at[idx_v.at[slot, 0]], buf_v.at[b],
                             sgs[b])

        def g_wait(j, slot, b):
            pltpu.make_async_copy(hts_hbm.at[idx_v.at[slot, 0]], buf_v.at[b],
                                  sgs[b]).wait()

        def s_issue(j, slot, b):
            pltpu.async_copy(buf_v.at[b], acc_sh.at[idx_v.at[slot, 1]],
                             sss[b], add=True)

        def s_wait(j, slot, b):
            pltpu.make_async_copy(buf_v.at[b], acc_sh.at[idx_v.at[slot, 1]],
                                  sss[b]).wait()

        idx_issue(0, 0)
        idx_issue(1, 1)
        idx_issue(2, 2)
        idx_wait(0, 0)
        g_issue(0, 0, 0)

        def body(j, _):
            m4 = j % 4

            def stage(slot):
                b = slot % 2
                nslot = (slot + 1) % 4
                pslot = (slot + 3) % 4
                g_wait(j, slot, b)

                @pl.when(j >= 1)
                def _():
                    s_wait(j - 1, pslot, 1 - b)

                @pl.when(j + 3 < NCH)
                def _():
                    idx_issue(j + 3, pslot)

                @pl.when(j + 1 < NCH)
                def _():
                    idx_wait(j + 1, nslot)
                    g_issue(j + 1, nslot, 1 - b)

                s_issue(j, slot, b)

            for k in range(4):
                @pl.when(m4 == k)
                def _(k=k):
                    stage(k)

            return 0

        lax.fori_loop(0, NCH, body, 0)
        s_wait(NCH - 1, (NCH - 1) % 4, (NCH - 1) % 2)
        plsc.subcore_barrier()
        pltpu.sync_copy(acc_sh.at[pl.ds(s * RPT, RPT), :],
                        agg_hbm.at[c, pl.ds(s * RPT, RPT), :])

    return edge_kernel(hts, comb)


def _tc_pre_body(x_r, degp_r, bg_r, bb_r, bm_r, bv_r, gw_r, gb_r, fw_r, fb_r,
                 hts_r, base_r, dinv_r):
    scale = bg_r[...] * lax.rsqrt(bv_r[...] + EPS)
    inp = jnp.maximum((x_r[...] - bm_r[...]) * scale + bb_r[...], 0.0)
    ht = jnp.dot(inp, gw_r[...], preferred_element_type=jnp.float32)
    deg = degp_r[0] + degp_r[1] + 1.0
    dinv = lax.rsqrt(deg)
    row = pl.program_id(0) * RB + lax.broadcasted_iota(jnp.int32, (RB, 1), 0)
    hts_r[...] = jnp.where(row < N, ht * dinv, 0.0)
    base_r[...] = inp + jnp.dot(inp, fw_r[...],
                                preferred_element_type=jnp.float32) \
        + fb_r[...] + gb_r[...]
    dinv_r[...] = dinv


def _tc_pre(x, degp3, bn_gamma, bn_beta, bn_mean, bn_var,
            gcn_W, gcn_b, fc1_W, fc1_b):
    vec = pl.BlockSpec((1, D), lambda j: (0, 0))
    mat = pl.BlockSpec((D, D), lambda j: (0, 0))
    return pl.pallas_call(
        _tc_pre_body,
        grid=(GRID,),
        in_specs=[
            pl.BlockSpec((RB, D), lambda j: (j, 0)),
            pl.BlockSpec((NC, RB, 1), lambda j: (0, j, 0)),
            vec, vec, vec, vec, mat, vec, mat, vec,
        ],
        out_specs=[
            pl.BlockSpec((RB, D), lambda j: (j, 0)),
            pl.BlockSpec((RB, D), lambda j: (j, 0)),
            pl.BlockSpec((RB, 1), lambda j: (j, 0)),
        ],
        out_shape=[
            jax.ShapeDtypeStruct((NPAD, D), jnp.float32),
            jax.ShapeDtypeStruct((NPAD, D), jnp.float32),
            jax.ShapeDtypeStruct((NPAD, 1), jnp.float32),
        ],
    )(x, degp3, bn_gamma, bn_beta, bn_mean, bn_var, gcn_W, gcn_b, fc1_W, fc1_b)


def _tc_post_body(aggp_r, hts_r, base_r, dinv_r, ow_r, ob_r, fin_r):
    agg = aggp_r[0] + aggp_r[1] + hts_r[...]
    node1 = dinv_r[...] * agg + base_r[...]
    fin_r[...] = jnp.dot(jnp.maximum(node1, 0.0), ow_r[...],
                         preferred_element_type=jnp.float32) + ob_r[...]


def _tc_post(aggp, hts, base, dinv, out_W, out_b):
    return pl.pallas_call(
        _tc_post_body,
        grid=(GRID,),
        in_specs=[
            pl.BlockSpec((NC, RB, D), lambda j: (0, j, 0)),
            pl.BlockSpec((RB, D), lambda j: (j, 0)),
            pl.BlockSpec((RB, D), lambda j: (j, 0)),
            pl.BlockSpec((RB, 1), lambda j: (j, 0)),
            pl.BlockSpec((D, D), lambda j: (0, 0)),
            pl.BlockSpec((1, D), lambda j: (0, 0)),
        ],
        out_specs=pl.BlockSpec((RB, D), lambda j: (j, 0)),
        out_shape=jax.ShapeDtypeStruct((NPAD, D), jnp.float32),
    )(aggp, hts, base, dinv, out_W, out_b)


def kernel(x, edge_index, bn_gamma, bn_beta, bn_mean, bn_var,
           gcn_W, gcn_b, fc1_W, fc1_b, out_W, out_b):
    pad = jnp.full((NW, EPWP - EPW), NPAD - 1, dtype=jnp.int32)
    srcp = jnp.concatenate([edge_index[0].reshape(NW, EPW), pad], axis=1)
    dstp = jnp.concatenate([edge_index[1].reshape(NW, EPW), pad], axis=1)
    comb = jnp.stack([srcp.reshape(NW, NCH, K),
                      dstp.reshape(NW, NCH, K)], axis=2)

    degp = _sc_degree(comb)
    degp3 = degp.reshape(NC, NPAD, 1)

    hts, base, dinv = _tc_pre(
        x, degp3,
        bn_gamma.reshape(1, D), bn_beta.reshape(1, D),
        bn_mean.reshape(1, D), bn_var.reshape(1, D),
        gcn_W, gcn_b.reshape(1, D), fc1_W, fc1_b.reshape(1, D))

    aggp = _sc_scatter(hts, comb)

    return _tc_post(aggp, hts, base, dinv, out_W, out_b.reshape(1, D))[:N]
